# baseline (speedup 1.0000x reference)
import functools

import jax
import jax.numpy as jnp
from jax import lax
from jax.experimental import pallas as pl
from jax.experimental.pallas import tpu as pltpu
from jax.experimental.pallas import tpu_sc as plsc

NC, NS, LANES = 2, 16, 16
NW = NC * NS

B, L, D = 1024, 200, 128
N = B * L
NTOK_PER_W = N // NW
CHUNK = 128
NCHUNK = NTOK_PER_W // CHUNK
NGROUP = CHUNK // LANES

MAX_DEPTH, MAX_SIBLING, N_NODE_TYPES = 64, 256, 4

_INV_D = 1.0 / D
_EPS = 1e-5


def _rsqrt(v):
    bits = plsc.bitcast(v, jnp.int32)
    seed = jnp.full((LANES,), 0x5F3759DF, jnp.int32) - (bits >> 1)
    y = plsc.bitcast(seed, jnp.float32)
    half_v = v * 0.5
    for _ in range(3):
        y = y * (1.5 - half_v * y * y)
    return y


def _body(tok_hbm, nt_hbm, dep_hbm, sib_hbm, par_hbm,
          key_hbm, val_hbm, dept_hbm, sibt_hbm, ntt_hbm, part_hbm,
          gam_hbm, bet_hbm, out_hbm,
          tok_i, nt_i, dep_i, sib_i, par_i,
          keyrows, valrows, parrows, xbuf,
          dep_t, sib_t, nt_t, gam_v, bet_v, sem):
    wid = lax.axis_index("s") * NC + lax.axis_index("c")
    wbase = wid * NTOK_PER_W

    pltpu.sync_copy(dept_hbm, dep_t)
    pltpu.sync_copy(sibt_hbm, sib_t)
    pltpu.sync_copy(ntt_hbm, nt_t)
    pltpu.sync_copy(gam_hbm, gam_v)
    pltpu.sync_copy(bet_hbm, bet_v)

    iota = lax.iota(jnp.int32, LANES)

    def chunk_body(c, carry):
        base = wbase + c * CHUNK
        sl = pl.ds(base, CHUNK)
        pltpu.sync_copy(tok_hbm.at[sl], tok_i)
        pltpu.sync_copy(nt_hbm.at[sl], nt_i)
        pltpu.sync_copy(dep_hbm.at[sl], dep_i)
        pltpu.sync_copy(sib_hbm.at[sl], sib_i)
        pltpu.sync_copy(par_hbm.at[sl], par_i)

        d1 = pltpu.async_copy(key_hbm.at[tok_i], keyrows, sem)
        d2 = pltpu.async_copy(val_hbm.at[tok_i], valrows, sem)
        d3 = pltpu.async_copy(part_hbm.at[par_i], parrows, sem)
        d1.wait()
        d2.wait()
        d3.wait()

        for g in range(NGROUP):
            rows = iota + (g * LANES)
            gsl = pl.ds(g * LANES, LANES)
            ntg = nt_i[gsl]
            depg = dep_i[gsl]
            sibg = sib_i[gsl]
            is_key = (ntg == 0) | (ntg == 2)

            def accum(d, sc):
                s, ss = sc
                cols = jnp.full((LANES,), d, jnp.int32)
                kd = plsc.load_gather(keyrows, [rows, cols])
                vd = plsc.load_gather(valrows, [rows, cols])
                pd = plsc.load_gather(parrows, [rows, cols])
                dd = plsc.load_gather(dep_t, [depg, cols])
                sd = plsc.load_gather(sib_t, [sibg, cols])
                nd = plsc.load_gather(nt_t, [ntg, cols])
                x = jnp.where(is_key, kd, vd) + pd + dd + sd + nd
                plsc.store_scatter(xbuf, [rows, cols], x)
                return s + x, ss + x * x

            zero = jnp.zeros((LANES,), jnp.float32)
            s, ss = lax.fori_loop(0, D, accum, (zero, zero))
            mu = s * _INV_D
            var = ss * _INV_D - mu * mu
            a = _rsqrt(var + _EPS)
            b = -mu * a

            def norm(d, _):
                cols = jnp.full((LANES,), d, jnp.int32)
                xg = plsc.load_gather(xbuf, [rows, cols])
                gm = plsc.load_gather(gam_v, [cols])
                bt = plsc.load_gather(bet_v, [cols])
                y = (xg * a + b) * gm + bt
                plsc.store_scatter(xbuf, [rows, cols], y)
                return 0

            lax.fori_loop(0, D, norm, 0)

        pltpu.sync_copy(xbuf, out_hbm.at[sl])
        return carry

    lax.fori_loop(0, NCHUNK, chunk_body, 0)


@jax.jit
def _sc_embed(tok, nt, dep, sib, par, key_table, value_table, depth_table,
              sibling_table, node_type_table, parent_key_table,
              ln_gamma, ln_beta):
    mesh = plsc.VectorSubcoreMesh(core_axis_name="c", subcore_axis_name="s",
                                  num_cores=NC, num_subcores=NS)
    fn = pl.kernel(
        _body,
        out_type=jax.ShapeDtypeStruct((N, D), jnp.float32),
        mesh=mesh,
        compiler_params=pltpu.CompilerParams(needs_layout_passes=False),
        scratch_types=[
            pltpu.VMEM((CHUNK,), jnp.int32),
            pltpu.VMEM((CHUNK,), jnp.int32),
            pltpu.VMEM((CHUNK,), jnp.int32),
            pltpu.VMEM((CHUNK,), jnp.int32),
            pltpu.VMEM((CHUNK,), jnp.int32),
            pltpu.VMEM((CHUNK, D), jnp.float32),
            pltpu.VMEM((CHUNK, D), jnp.float32),
            pltpu.VMEM((CHUNK, D), jnp.float32),
            pltpu.VMEM((CHUNK, D), jnp.float32),
            pltpu.VMEM((MAX_DEPTH, D), jnp.float32),
            pltpu.VMEM((MAX_SIBLING, D), jnp.float32),
            pltpu.VMEM((N_NODE_TYPES, D), jnp.float32),
            pltpu.VMEM((D,), jnp.float32),
            pltpu.VMEM((D,), jnp.float32),
            pltpu.SemaphoreType.DMA,
        ],
    )
    return fn(tok, nt, dep, sib, par, key_table, value_table, depth_table,
              sibling_table, node_type_table, parent_key_table,
              ln_gamma, ln_beta)


def kernel(token_ids, node_types, depths, sibling_indices, parent_key_ids,
           key_table, value_table, depth_table, sibling_table,
           node_type_table, parent_key_table, ln_gamma, ln_beta):
    tok = token_ids.reshape(N).astype(jnp.int32)
    nt = node_types.reshape(N).astype(jnp.int32)
    dep = depths.reshape(N).astype(jnp.int32)
    sib = sibling_indices.reshape(N).astype(jnp.int32)
    par = parent_key_ids.reshape(N).astype(jnp.int32)
    out = _sc_embed(tok, nt, dep, sib, par,
                    key_table, value_table, depth_table, sibling_table,
                    node_type_table, parent_key_table,
                    ln_gamma.astype(jnp.float32), ln_beta.astype(jnp.float32))
    return out.reshape(B, L, D)


# speedup vs baseline: 5.7548x; 5.7548x over previous
import functools

import jax
import jax.numpy as jnp
from jax import lax
from jax.experimental import pallas as pl
from jax.experimental.pallas import tpu as pltpu
from jax.experimental.pallas import tpu_sc as plsc

NC, NS, LANES = 2, 16, 16
NW = NC * NS

B, L, D = 1024, 200, 128
N = B * L
NTOK_PER_W = N // NW
CHUNK = 128
NCHUNK = NTOK_PER_W // CHUNK
NGROUP = CHUNK // LANES

MAX_DEPTH, MAX_SIBLING, N_NODE_TYPES = 64, 256, 4

_INV_D = 1.0 / D
_EPS = 1e-5


def _rsqrt(v):
    bits = plsc.bitcast(v, jnp.int32)
    seed = jnp.full((LANES,), 0x5F3759DF, jnp.int32) - (bits >> 1)
    y = plsc.bitcast(seed, jnp.float32)
    half_v = v * 0.5
    for _ in range(3):
        y = y * (1.5 - half_v * y * y)
    return y


NJ = D // LANES
UNROLL = 2


def _body(tok_hbm, nt_hbm, dep_hbm, sib_hbm, par_hbm,
          key_hbm, val_hbm, dept_hbm, sibt_hbm, ntt_hbm, part_hbm,
          gam_hbm, bet_hbm, out_hbm,
          tok_i, nt_i, dep_i, sib_i, par_i,
          keyrows, valrows, parrows, xbuf, isk_f,
          dep_t, sib_t, nt_t, gam_v, bet_v, sem):
    wid = lax.axis_index("s") * NC + lax.axis_index("c")
    wbase = wid * NTOK_PER_W

    pltpu.sync_copy(dept_hbm, dep_t)
    pltpu.sync_copy(sibt_hbm, sib_t)
    pltpu.sync_copy(ntt_hbm, nt_t)
    pltpu.sync_copy(gam_hbm, gam_v)
    pltpu.sync_copy(bet_hbm, bet_v)

    iota = lax.iota(jnp.int32, LANES)
    colv = [iota + (LANES * j) for j in range(NJ)]
    gms = [gam_v[pl.ds(LANES * j, LANES)] for j in range(NJ)]
    bts = [bet_v[pl.ds(LANES * j, LANES)] for j in range(NJ)]

    def chunk_body(c, carry):
        base = wbase + c * CHUNK
        sl = pl.ds(base, CHUNK)
        pltpu.sync_copy(tok_hbm.at[sl], tok_i)
        pltpu.sync_copy(nt_hbm.at[sl], nt_i)
        pltpu.sync_copy(dep_hbm.at[sl], dep_i)
        pltpu.sync_copy(sib_hbm.at[sl], sib_i)
        pltpu.sync_copy(par_hbm.at[sl], par_i)

        d1 = pltpu.async_copy(key_hbm.at[tok_i], keyrows, sem)
        d2 = pltpu.async_copy(val_hbm.at[tok_i], valrows, sem)
        d3 = pltpu.async_copy(part_hbm.at[par_i], parrows, sem)

        for g in range(NGROUP):
            gsl = pl.ds(g * LANES, LANES)
            ntg = nt_i[gsl]
            isk_f[gsl] = jnp.where((ntg == 0) | (ntg == 2), 1.0, 0.0)

        d1.wait()
        d2.wait()
        d3.wait()

        @plsc.parallel_loop(0, CHUNK, 1, unroll=UNROLL)
        def token_body(t):
            tsp = jnp.full((LANES,), t, jnp.int32)
            m = plsc.load_gather(isk_f, [tsp])
            dsp = plsc.load_gather(dep_i, [tsp])
            ssp = plsc.load_gather(sib_i, [tsp])
            nsp = plsc.load_gather(nt_i, [tsp])
            xs = []
            s = jnp.zeros((LANES,), jnp.float32)
            ss = jnp.zeros((LANES,), jnp.float32)
            for j in range(NJ):
                kd = plsc.load_gather(keyrows, [tsp, colv[j]])
                vd = plsc.load_gather(valrows, [tsp, colv[j]])
                pd = plsc.load_gather(parrows, [tsp, colv[j]])
                dd = plsc.load_gather(dep_t, [dsp, colv[j]])
                sd = plsc.load_gather(sib_t, [ssp, colv[j]])
                nd = plsc.load_gather(nt_t, [nsp, colv[j]])
                x = vd + (kd - vd) * m + pd + dd + sd + nd
                xs.append(x)
                s = s + x
                ss = ss + x * x
            mu = jnp.full((LANES,), jnp.sum(s)) * _INV_D
            msq = jnp.full((LANES,), jnp.sum(ss)) * _INV_D
            a = _rsqrt(msq - mu * mu + _EPS)
            b = -mu * a
            for j in range(NJ):
                y = (xs[j] * a + b) * gms[j] + bts[j]
                plsc.store_scatter(xbuf, [tsp, colv[j]], y)

        pltpu.sync_copy(xbuf, out_hbm.at[sl])
        return carry

    lax.fori_loop(0, NCHUNK, chunk_body, 0)


@jax.jit
def _sc_embed(tok, nt, dep, sib, par, key_table, value_table, depth_table,
              sibling_table, node_type_table, parent_key_table,
              ln_gamma, ln_beta):
    mesh = plsc.VectorSubcoreMesh(core_axis_name="c", subcore_axis_name="s",
                                  num_cores=NC, num_subcores=NS)
    fn = pl.kernel(
        _body,
        out_type=jax.ShapeDtypeStruct((N, D), jnp.float32),
        mesh=mesh,
        compiler_params=pltpu.CompilerParams(needs_layout_passes=False),
        scratch_types=[
            pltpu.VMEM((CHUNK,), jnp.int32),
            pltpu.VMEM((CHUNK,), jnp.int32),
            pltpu.VMEM((CHUNK,), jnp.int32),
            pltpu.VMEM((CHUNK,), jnp.int32),
            pltpu.VMEM((CHUNK,), jnp.int32),
            pltpu.VMEM((CHUNK, D), jnp.float32),
            pltpu.VMEM((CHUNK, D), jnp.float32),
            pltpu.VMEM((CHUNK, D), jnp.float32),
            pltpu.VMEM((CHUNK, D), jnp.float32),
            pltpu.VMEM((CHUNK,), jnp.float32),
            pltpu.VMEM((MAX_DEPTH, D), jnp.float32),
            pltpu.VMEM((MAX_SIBLING, D), jnp.float32),
            pltpu.VMEM((N_NODE_TYPES, D), jnp.float32),
            pltpu.VMEM((D,), jnp.float32),
            pltpu.VMEM((D,), jnp.float32),
            pltpu.SemaphoreType.DMA,
        ],
    )
    return fn(tok, nt, dep, sib, par, key_table, value_table, depth_table,
              sibling_table, node_type_table, parent_key_table,
              ln_gamma, ln_beta)


def kernel(token_ids, node_types, depths, sibling_indices, parent_key_ids,
           key_table, value_table, depth_table, sibling_table,
           node_type_table, parent_key_table, ln_gamma, ln_beta):
    tok = token_ids.reshape(N).astype(jnp.int32)
    nt = node_types.reshape(N).astype(jnp.int32)
    dep = depths.reshape(N).astype(jnp.int32)
    sib = sibling_indices.reshape(N).astype(jnp.int32)
    par = parent_key_ids.reshape(N).astype(jnp.int32)
    out = _sc_embed(tok, nt, dep, sib, par,
                    key_table, value_table, depth_table, sibling_table,
                    node_type_table, parent_key_table,
                    ln_gamma.astype(jnp.float32), ln_beta.astype(jnp.float32))
    return out.reshape(B, L, D)


# speedup vs baseline: 7.1137x; 1.2361x over previous
import functools

import jax
import jax.numpy as jnp
from jax import lax
from jax.experimental import pallas as pl
from jax.experimental.pallas import tpu as pltpu
from jax.experimental.pallas import tpu_sc as plsc

NC, NS, LANES = 2, 16, 16
NW = NC * NS

B, L, D = 1024, 200, 128
N = B * L
NTOK_PER_W = N // NW
CHUNK = 128
NCHUNK = NTOK_PER_W // CHUNK
NGROUP = CHUNK // LANES

MAX_DEPTH, MAX_SIBLING, N_NODE_TYPES = 64, 256, 4

_INV_D = 1.0 / D
_EPS = 1e-5


def _rsqrt(v):
    bits = plsc.bitcast(v, jnp.int32)
    seed = jnp.full((LANES,), 0x5F3759DF, jnp.int32) - (bits >> 1)
    y = plsc.bitcast(seed, jnp.float32)
    half_v = v * 0.5
    for _ in range(3):
        y = y * (1.5 - half_v * y * y)
    return y


NJ = D // LANES
UNROLL = 4


def _body(tok_hbm, nt_hbm, dep_hbm, sib_hbm, par_hbm,
          key_hbm, val_hbm, dept_hbm, sibt_hbm, ntt_hbm, part_hbm,
          gam_hbm, bet_hbm, out_hbm,
          tok_i, nt_i, dep_i, sib_i, par_i,
          keyrows, valrows, parrows, xbuf, isk_f,
          dep_t, sib_t, nt_t, gam_v, bet_v, sem):
    wid = lax.axis_index("s") * NC + lax.axis_index("c")
    wbase = wid * NTOK_PER_W

    pltpu.sync_copy(dept_hbm, dep_t)
    pltpu.sync_copy(sibt_hbm, sib_t)
    pltpu.sync_copy(ntt_hbm, nt_t)
    pltpu.sync_copy(gam_hbm, gam_v)
    pltpu.sync_copy(bet_hbm, bet_v)

    iota = lax.iota(jnp.int32, LANES)
    colv = [iota + (LANES * j) for j in range(NJ)]
    gms = [gam_v[pl.ds(LANES * j, LANES)] for j in range(NJ)]
    bts = [bet_v[pl.ds(LANES * j, LANES)] for j in range(NJ)]

    def chunk_body(c, carry):
        base = wbase + c * CHUNK
        sl = pl.ds(base, CHUNK)
        pltpu.sync_copy(tok_hbm.at[sl], tok_i)
        pltpu.sync_copy(nt_hbm.at[sl], nt_i)
        pltpu.sync_copy(dep_hbm.at[sl], dep_i)
        pltpu.sync_copy(sib_hbm.at[sl], sib_i)
        pltpu.sync_copy(par_hbm.at[sl], par_i)

        d1 = pltpu.async_copy(key_hbm.at[tok_i], keyrows, sem)
        d2 = pltpu.async_copy(val_hbm.at[tok_i], valrows, sem)
        d3 = pltpu.async_copy(part_hbm.at[par_i], parrows, sem)

        for g in range(NGROUP):
            gsl = pl.ds(g * LANES, LANES)
            ntg = nt_i[gsl]
            isk_f[gsl] = jnp.where((ntg == 0) | (ntg == 2), 1.0, 0.0)

        d1.wait()
        d2.wait()
        d3.wait()

        @plsc.parallel_loop(0, CHUNK, 1, unroll=UNROLL)
        def token_body(t):
            tsp = jnp.full((LANES,), t, jnp.int32)
            m = plsc.load_gather(isk_f, [tsp])
            dsp = plsc.load_gather(dep_i, [tsp])
            ssp = plsc.load_gather(sib_i, [tsp])
            nsp = plsc.load_gather(nt_i, [tsp])
            xs = []
            s = jnp.zeros((LANES,), jnp.float32)
            ss = jnp.zeros((LANES,), jnp.float32)
            for j in range(NJ):
                kd = plsc.load_gather(keyrows, [tsp, colv[j]])
                vd = plsc.load_gather(valrows, [tsp, colv[j]])
                pd = plsc.load_gather(parrows, [tsp, colv[j]])
                dd = plsc.load_gather(dep_t, [dsp, colv[j]])
                sd = plsc.load_gather(sib_t, [ssp, colv[j]])
                nd = plsc.load_gather(nt_t, [nsp, colv[j]])
                x = vd + (kd - vd) * m + pd + dd + sd + nd
                xs.append(x)
                s = s + x
                ss = ss + x * x
            mu = jnp.full((LANES,), jnp.sum(s)) * _INV_D
            msq = jnp.full((LANES,), jnp.sum(ss)) * _INV_D
            a = _rsqrt(msq - mu * mu + _EPS)
            b = -mu * a
            for j in range(NJ):
                y = (xs[j] * a + b) * gms[j] + bts[j]
                plsc.store_scatter(xbuf, [tsp, colv[j]], y)

        pltpu.sync_copy(xbuf, out_hbm.at[sl])
        return carry

    lax.fori_loop(0, NCHUNK, chunk_body, 0)


@jax.jit
def _sc_embed(tok, nt, dep, sib, par, key_table, value_table, depth_table,
              sibling_table, node_type_table, parent_key_table,
              ln_gamma, ln_beta):
    mesh = plsc.VectorSubcoreMesh(core_axis_name="c", subcore_axis_name="s",
                                  num_cores=NC, num_subcores=NS)
    fn = pl.kernel(
        _body,
        out_type=jax.ShapeDtypeStruct((N, D), jnp.float32),
        mesh=mesh,
        compiler_params=pltpu.CompilerParams(needs_layout_passes=False),
        scratch_types=[
            pltpu.VMEM((CHUNK,), jnp.int32),
            pltpu.VMEM((CHUNK,), jnp.int32),
            pltpu.VMEM((CHUNK,), jnp.int32),
            pltpu.VMEM((CHUNK,), jnp.int32),
            pltpu.VMEM((CHUNK,), jnp.int32),
            pltpu.VMEM((CHUNK, D), jnp.float32),
            pltpu.VMEM((CHUNK, D), jnp.float32),
            pltpu.VMEM((CHUNK, D), jnp.float32),
            pltpu.VMEM((CHUNK, D), jnp.float32),
            pltpu.VMEM((CHUNK,), jnp.float32),
            pltpu.VMEM((MAX_DEPTH, D), jnp.float32),
            pltpu.VMEM((MAX_SIBLING, D), jnp.float32),
            pltpu.VMEM((N_NODE_TYPES, D), jnp.float32),
            pltpu.VMEM((D,), jnp.float32),
            pltpu.VMEM((D,), jnp.float32),
            pltpu.SemaphoreType.DMA,
        ],
    )
    return fn(tok, nt, dep, sib, par, key_table, value_table, depth_table,
              sibling_table, node_type_table, parent_key_table,
              ln_gamma, ln_beta)


def kernel(token_ids, node_types, depths, sibling_indices, parent_key_ids,
           key_table, value_table, depth_table, sibling_table,
           node_type_table, parent_key_table, ln_gamma, ln_beta):
    tok = token_ids.reshape(N).astype(jnp.int32)
    nt = node_types.reshape(N).astype(jnp.int32)
    dep = depths.reshape(N).astype(jnp.int32)
    sib = sibling_indices.reshape(N).astype(jnp.int32)
    par = parent_key_ids.reshape(N).astype(jnp.int32)
    out = _sc_embed(tok, nt, dep, sib, par,
                    key_table, value_table, depth_table, sibling_table,
                    node_type_table, parent_key_table,
                    ln_gamma.astype(jnp.float32), ln_beta.astype(jnp.float32))
    return out.reshape(B, L, D)
